# Initial kernel scaffold; baseline (speedup 1.0000x reference)
import jax
import jax.numpy as jnp
from jax.experimental import pallas as pl


def kernel(x, pos, edge_index, edge_attr, W_enc, b_enc, We_enc, be_enc, msg_W1, msg_b1, msg_W2, msg_b2, coord_W1, coord_b1, coord_W2, coord_b2, node_W1, node_b1, node_W2, node_b2, ro_W1, ro_b1, ro_W2, ro_b2):
    raise NotImplementedError("write your pallas kernel here")


# speedup vs baseline: 1.9953x; 1.9953x over previous
import functools

import jax
import jax.numpy as jnp
from jax import lax
from jax.experimental import pallas as pl
from jax.experimental.pallas import tpu as pltpu
from jax.experimental.pallas import tpu_sc as plsc

N = 10000
E = 160000
H = 128
DE = 16
PW = 16
L = 4

NC = 2
NS = 16
NW = NC * NS
EPT = E // NW
EPT1 = E // NS
C = 40
RB = 624
RLS = NS * RB
RTAIL = N - RLS

f32 = jnp.float32

_MESH = plsc.VectorSubcoreMesh(
    core_axis_name="c", subcore_axis_name="s", num_cores=NC, num_subcores=NS
)


def _silu(a):
    return a * jax.nn.sigmoid(a)


def _enc_proj_body(x_ref, We_ref, be_ref, W1a_ref, W1b_ref,
                   h_ref, hA_ref, hB_ref):
    h = jnp.dot(x_ref[...], We_ref[...], preferred_element_type=f32) + be_ref[...]
    h_ref[...] = h
    hA_ref[...] = jnp.dot(h, W1a_ref[...], preferred_element_type=f32)
    hB_ref[...] = jnp.dot(h, W1b_ref[...], preferred_element_type=f32)


def _tc_encode_proj(x, W_enc, b_enc2, W1a, W1b):
    BN = 1000
    wspec = pl.BlockSpec((H, H), lambda i: (0, 0))
    return pl.pallas_call(
        _enc_proj_body,
        grid=(N // BN,),
        in_specs=[
            pl.BlockSpec((BN, H), lambda i: (i, 0)),
            wspec,
            pl.BlockSpec((1, H), lambda i: (0, 0)),
            wspec,
            wspec,
        ],
        out_specs=[pl.BlockSpec((BN, H), lambda i: (i, 0))] * 3,
        out_shape=[jax.ShapeDtypeStruct((N, H), f32)] * 3,
    )(x, W_enc, b_enc2, W1a, W1b)


def _message_body(g_ref, d_ref, ea_ref, W1cd_ref, b1_ref,
                  Wee_ref, bee_ref, W2_ref, b2_ref, m2_ref, cdn_ref):
    w1c = W1cd_ref[0:1, :]
    Wd = W1cd_ref[1:, :]
    Wfold = jnp.dot(Wee_ref[...], Wd, preferred_element_type=f32)
    b1f = b1_ref[...] + jnp.dot(bee_ref[...], Wd, preferred_element_type=f32)
    d = d_ref[...]
    dist = jnp.sqrt(jnp.sum(d * d, axis=1, keepdims=True))
    m1 = (g_ref[...] + dist * w1c
          + jnp.dot(ea_ref[...], Wfold, preferred_element_type=f32) + b1f)
    m = _silu(m1)
    m2_ref[...] = _silu(jnp.dot(m, W2_ref[...], preferred_element_type=f32)
                        + b2_ref[...])
    cdn_ref[...] = -d / (dist + 1e-8)


def _tc_message(g, d16, ea, W1cd, b1_2, We_enc, be_enc2, W2, b2_2):
    BE = 1000
    const = lambda shape: pl.BlockSpec(shape, lambda i: (0, 0))
    return pl.pallas_call(
        _message_body,
        grid=(E // BE,),
        in_specs=[
            pl.BlockSpec((BE, H), lambda i: (i, 0)),
            pl.BlockSpec((BE, PW), lambda i: (i, 0)),
            pl.BlockSpec((BE, DE), lambda i: (i, 0)),
            const((H + 1, H)),
            const((1, H)),
            const((DE, H)),
            const((1, H)),
            const((H, H)),
            const((1, H)),
        ],
        out_specs=[
            pl.BlockSpec((BE, H), lambda i: (i, 0)),
            pl.BlockSpec((BE, PW), lambda i: (i, 0)),
        ],
        out_shape=[
            jax.ShapeDtypeStruct((E, H), f32),
            jax.ShapeDtypeStruct((E, PW), f32),
        ],
    )(g, d16, ea, W1cd, b1_2, We_enc, be_enc2, W2, b2_2)


def _update_body(p0_ref, p1_ref, h_ref, nW1a_ref, nW1b_ref, nb1_ref,
                 nW2_ref, nb2_ref, cW1_ref, cb1_ref, cW2t_ref, cb2_ref,
                 W1a_ref, W1b_ref, hn_ref, hA_ref, hB_ref, cm_ref):
    agg = p0_ref[...] + p1_ref[...]
    u = _silu(jnp.dot(h_ref[...], nW1a_ref[...], preferred_element_type=f32)
              + jnp.dot(agg, nW1b_ref[...], preferred_element_type=f32)
              + nb1_ref[...])
    hn = jnp.dot(u, nW2_ref[...], preferred_element_type=f32) + nb2_ref[...]
    hn_ref[...] = hn
    hA_ref[...] = jnp.dot(hn, W1a_ref[...], preferred_element_type=f32)
    hB_ref[...] = jnp.dot(hn, W1b_ref[...], preferred_element_type=f32)
    t = _silu(jnp.dot(agg, cW1_ref[...], preferred_element_type=f32)
              + cb1_ref[...])
    cm = jnp.dot(t, cW2t_ref[...], preferred_element_type=f32) + cb2_ref[...]
    cm_ref[...] = cm[:, :PW]


def _update_last_body(p0_ref, p1_ref, h_ref, nW1a_ref, nW1b_ref, nb1_ref,
                      nW2_ref, nb2_ref, hn_ref):
    agg = p0_ref[...] + p1_ref[...]
    u = _silu(jnp.dot(h_ref[...], nW1a_ref[...], preferred_element_type=f32)
              + jnp.dot(agg, nW1b_ref[...], preferred_element_type=f32)
              + nb1_ref[...])
    hn_ref[...] = jnp.dot(u, nW2_ref[...], preferred_element_type=f32) + nb2_ref[...]


def _tc_update(parts, h, nW1a, nW1b, nb1_2, nW2, nb2_2,
               cW1, cb1_2, cW2t, cb2t, W1a, W1b):
    BN = 1000
    wspec = pl.BlockSpec((H, H), lambda i: (0, 0))
    bspec = pl.BlockSpec((1, H), lambda i: (0, 0))
    nspec = pl.BlockSpec((BN, H), lambda i: (i, 0))
    return pl.pallas_call(
        _update_body,
        grid=(N // BN,),
        in_specs=[
            pl.BlockSpec((BN, H), lambda i: (i, 0)),
            pl.BlockSpec((BN, H), lambda i: (i + N // BN, 0)),
            nspec, wspec, wspec, bspec, wspec, bspec,
            wspec, bspec, wspec, bspec, wspec, wspec,
        ],
        out_specs=[nspec, nspec, nspec, pl.BlockSpec((BN, PW), lambda i: (i, 0))],
        out_shape=[
            jax.ShapeDtypeStruct((N, H), f32),
            jax.ShapeDtypeStruct((N, H), f32),
            jax.ShapeDtypeStruct((N, H), f32),
            jax.ShapeDtypeStruct((N, PW), f32),
        ],
    )(parts, parts, h, nW1a, nW1b, nb1_2, nW2, nb2_2,
      cW1, cb1_2, cW2t, cb2t, W1a, W1b)


def _tc_update_last(parts, h, nW1a, nW1b, nb1_2, nW2, nb2_2):
    BN = 1000
    wspec = pl.BlockSpec((H, H), lambda i: (0, 0))
    bspec = pl.BlockSpec((1, H), lambda i: (0, 0))
    nspec = pl.BlockSpec((BN, H), lambda i: (i, 0))
    return pl.pallas_call(
        _update_last_body,
        grid=(N // BN,),
        in_specs=[
            pl.BlockSpec((BN, H), lambda i: (i, 0)),
            pl.BlockSpec((BN, H), lambda i: (i + N // BN, 0)),
            nspec, wspec, wspec, bspec, wspec, bspec,
        ],
        out_specs=nspec,
        out_shape=jax.ShapeDtypeStruct((N, H), f32),
    )(parts, parts, h, nW1a, nW1b, nb1_2, nW2, nb2_2)


def _readout_body(h_ref, W1_ref, b1_ref, w2_ref, b2_ref, out_ref):
    hg = jnp.mean(h_ref[...], axis=0, keepdims=True)
    hg8 = jnp.broadcast_to(hg, (8, H))
    r = jnp.maximum(
        jnp.dot(hg8, W1_ref[...], preferred_element_type=f32) + b1_ref[...], 0.0)
    o = jnp.sum(r * w2_ref[...], axis=1, keepdims=True) + b2_ref[...]
    out_ref[...] = o[0:1, :]


def _tc_readout(h, ro_W1, ro_b1_2, ro_w2r, ro_b2_2):
    return pl.pallas_call(
        _readout_body,
        out_shape=jax.ShapeDtypeStruct((1, 1), f32),
    )(h, ro_W1, ro_b1_2, ro_w2r, ro_b2_2)


def _stage_rows(s, src_hbm, dst_sh, buf):
    for k in range(RB // C):
        sl = pl.ds(s * RB + k * C, C)
        pltpu.sync_copy(src_hbm.at[sl], buf)
        pltpu.sync_copy(buf, dst_sh.at[sl])
    rem = RB - (RB // C) * C
    if rem:
        sl = pl.ds(s * RB + (RB // C) * C, rem)
        bsl = buf.at[pl.ds(0, rem)]
        pltpu.sync_copy(src_hbm.at[sl], bsl)
        pltpu.sync_copy(bsl, dst_sh.at[sl])

    @pl.when(s == NS - 1)
    def _():
        sl = pl.ds(RLS, RTAIL)
        bsl = buf.at[pl.ds(0, RTAIL)]
        pltpu.sync_copy(src_hbm.at[sl], bsl)
        pltpu.sync_copy(bsl, dst_sh.at[sl])


def _unstage_rows(s, src_sh, dst_hbm, buf, dst_off=0):
    for k in range(RB // C):
        sl = pl.ds(s * RB + k * C, C)
        pltpu.sync_copy(src_sh.at[sl], buf)
        pltpu.sync_copy(buf, dst_hbm.at[pl.ds(dst_off + s * RB + k * C, C)])
    rem = RB - (RB // C) * C
    if rem:
        sl = pl.ds(s * RB + (RB // C) * C, rem)
        bsl = buf.at[pl.ds(0, rem)]
        pltpu.sync_copy(src_sh.at[sl], bsl)
        pltpu.sync_copy(bsl, dst_hbm.at[pl.ds(dst_off + s * RB + (RB // C) * C, rem)])

    @pl.when(s == NS - 1)
    def _():
        bsl = buf.at[pl.ds(0, RTAIL)]
        pltpu.sync_copy(src_sh.at[pl.ds(RLS, RTAIL)], bsl)
        pltpu.sync_copy(bsl, dst_hbm.at[pl.ds(dst_off + RLS, RTAIL)])


def _sc_gpos_body(pos16, cdn, cmsg16, hA, hB, row, col,
                  g_out, d_out, pos_out,
                  idxr, idxc, bufA, bufB, bufPC, bufPR, pos_sh, cmsg_sh, sem):
    c = lax.axis_index("c")
    s = lax.axis_index("s")

    _stage_rows(s, pos16, pos_sh, bufPC)
    _stage_rows(s, cmsg16, cmsg_sh, bufPC)
    plsc.subcore_barrier()

    def sbody(j, carry):
        base = s * EPT1 + j * C
        pltpu.sync_copy(row.at[pl.ds(base, C)], idxr)
        pltpu.sync_copy(cdn.at[pl.ds(base, C)], bufPC)
        pltpu.sync_copy(cmsg_sh.at[idxr], bufPR)

        def mbody(i, carry2):
            bufPR[i, :] = bufPR[i, :] * bufPC[i, :]
            return carry2
        lax.fori_loop(0, C, mbody, 0)
        pltpu.sync_copy(bufPR, pos_sh.at[idxr], add=True)
        return carry

    lax.fori_loop(0, EPT1 // C, sbody, 0)
    plsc.subcore_barrier()

    @pl.when(c == 0)
    def _():
        _unstage_rows(s, pos_sh, pos_out, bufPC)

    base_e = (c * NS + s) * EPT

    def gbody(j, carry):
        base = base_e + j * C
        pltpu.sync_copy(row.at[pl.ds(base, C)], idxr)
        pltpu.sync_copy(col.at[pl.ds(base, C)], idxc)
        cpA = pltpu.async_copy(hA.at[idxc], bufA, sem)
        cpB = pltpu.async_copy(hB.at[idxr], bufB, sem)
        pltpu.sync_copy(pos_sh.at[idxc], bufPC)
        pltpu.sync_copy(pos_sh.at[idxr], bufPR)
        cpA.wait()
        cpB.wait()

        def rbody(i, carry2):
            bufPC[i, :] = bufPC[i, :] - bufPR[i, :]

            def cbody(k, carry3):
                sl = pl.ds(k * 16, 16)
                bufA[i, sl] = bufA[i, sl] + bufB[i, sl]
                return carry3
            return lax.fori_loop(0, H // 16, cbody, carry2)
        lax.fori_loop(0, C, rbody, 0)

        pltpu.sync_copy(bufA, g_out.at[pl.ds(base, C)])
        pltpu.sync_copy(bufPC, d_out.at[pl.ds(base, C)])
        return carry

    lax.fori_loop(0, EPT // C, gbody, 0)


_sc_gpos = functools.partial(
    pl.kernel,
    out_type=[
        jax.ShapeDtypeStruct((E, H), f32),
        jax.ShapeDtypeStruct((E, PW), f32),
        jax.ShapeDtypeStruct((N, PW), f32),
    ],
    mesh=_MESH,
    scratch_types=[
        pltpu.VMEM((C,), jnp.int32),
        pltpu.VMEM((C,), jnp.int32),
        pltpu.VMEM((C, H), f32),
        pltpu.VMEM((C, H), f32),
        pltpu.VMEM((C, PW), f32),
        pltpu.VMEM((C, PW), f32),
        pltpu.VMEM_SHARED((N, PW), f32),
        pltpu.VMEM_SHARED((N, PW), f32),
        pltpu.SemaphoreType.DMA,
    ],
)(_sc_gpos_body)


def _sc_segsum_body(m2, col, parts_out, idxb, buf, zbuf, agg_sh, sem):
    c = lax.axis_index("c")
    s = lax.axis_index("s")

    def zbody(i, carry):
        def zcol(k, carry2):
            zbuf[i, pl.ds(k * 16, 16)] = jnp.zeros((16,), f32)
            return carry2
        return lax.fori_loop(0, H // 16, zcol, carry)
    lax.fori_loop(0, RTAIL, zbody, 0)
    for k in range(RB // RTAIL):
        pltpu.sync_copy(zbuf, agg_sh.at[pl.ds(s * RB + k * RTAIL, RTAIL)])

    @pl.when(s == NS - 1)
    def _():
        pltpu.sync_copy(zbuf, agg_sh.at[pl.ds(RLS, RTAIL)])

    plsc.subcore_barrier()
    base_e = (c * NS + s) * EPT

    def body(j, carry):
        base = base_e + j * C
        pltpu.sync_copy(col.at[pl.ds(base, C)], idxb)
        pltpu.sync_copy(m2.at[pl.ds(base, C)], buf)
        pltpu.sync_copy(buf, agg_sh.at[idxb], add=True)
        return carry

    lax.fori_loop(0, EPT // C, body, 0)
    plsc.subcore_barrier()
    _unstage_rows(s, agg_sh, parts_out, buf, dst_off=c * N)


_sc_segsum = functools.partial(
    pl.kernel,
    out_type=jax.ShapeDtypeStruct((2 * N, H), f32),
    mesh=_MESH,
    scratch_types=[
        pltpu.VMEM((C,), jnp.int32),
        pltpu.VMEM((C, H), f32),
        pltpu.VMEM((RTAIL, H), f32),
        pltpu.VMEM_SHARED((N, H), f32),
        pltpu.SemaphoreType.DMA,
    ],
)(_sc_segsum_body)


def kernel(x, pos, edge_index, edge_attr, W_enc, b_enc, We_enc, be_enc,
           msg_W1, msg_b1, msg_W2, msg_b2, coord_W1, coord_b1, coord_W2,
           coord_b2, node_W1, node_b1, node_W2, node_b2,
           ro_W1, ro_b1, ro_W2, ro_b2):
    row = edge_index[0]
    col = edge_index[1]
    pos16 = jnp.pad(pos, ((0, 0), (0, PW - 3)))
    be_enc2 = be_enc.reshape(1, H)

    h, hA, hB = _tc_encode_proj(x, W_enc, b_enc.reshape(1, H),
                                msg_W1[0, :H], msg_W1[0, H:2 * H])

    cdn_prev = jnp.zeros((E, PW), f32)
    cmsg_prev = jnp.zeros((N, PW), f32)

    for l in range(L):
        g, d16, pos16 = _sc_gpos(pos16, cdn_prev, cmsg_prev, hA, hB, row, col)
        m2, cdn = _tc_message(g, d16, edge_attr,
                              msg_W1[l, 2 * H:], msg_b1[l].reshape(1, H),
                              We_enc, be_enc2,
                              msg_W2[l], msg_b2[l].reshape(1, H))
        parts = _sc_segsum(m2, col)
        if l < L - 1:
            h, hA, hB, cm16 = _tc_update(
                parts, h,
                node_W1[l, :H], node_W1[l, H:], node_b1[l].reshape(1, H),
                node_W2[l], node_b2[l].reshape(1, H),
                coord_W1[l], coord_b1[l].reshape(1, H),
                jnp.tile(coord_W2[l], (1, H)),
                jnp.tile(coord_b2[l].reshape(1, 1), (1, H)),
                msg_W1[l + 1, :H], msg_W1[l + 1, H:2 * H])
            cdn_prev, cmsg_prev = cdn, cm16
        else:
            h = _tc_update_last(
                parts, h,
                node_W1[l, :H], node_W1[l, H:], node_b1[l].reshape(1, H),
                node_W2[l], node_b2[l].reshape(1, H))

    return _tc_readout(h, ro_W1, ro_b1.reshape(1, H),
                       ro_W2.reshape(1, H), ro_b2.reshape(1, 1))


# speedup vs baseline: 2.1045x; 1.0548x over previous
import functools

import jax
import jax.numpy as jnp
from jax import lax
from jax.experimental import pallas as pl
from jax.experimental.pallas import tpu as pltpu
from jax.experimental.pallas import tpu_sc as plsc

N = 10000
E = 160000
H = 128
DE = 16
PW = 16
L = 4

NC = 2
NS = 16
NW = NC * NS
EPT = E // NW
EPT1 = E // NS
C = 40
CS = 40
RB = 624
RLS = NS * RB
RTAIL = N - RLS

f32 = jnp.float32

_MESH = plsc.VectorSubcoreMesh(
    core_axis_name="c", subcore_axis_name="s", num_cores=NC, num_subcores=NS
)


def _silu(a):
    return a * jax.nn.sigmoid(a)


def _enc_proj_body(x_ref, We_ref, be_ref, W1a_ref, W1b_ref,
                   h_ref, hA_ref, hB_ref):
    h = jnp.dot(x_ref[...], We_ref[...], preferred_element_type=f32) + be_ref[...]
    h_ref[...] = h
    hA_ref[...] = jnp.dot(h, W1a_ref[...], preferred_element_type=f32)
    hB_ref[...] = jnp.dot(h, W1b_ref[...], preferred_element_type=f32)


def _tc_encode_proj(x, W_enc, b_enc2, W1a, W1b):
    BN = 1000
    wspec = pl.BlockSpec((H, H), lambda i: (0, 0))
    return pl.pallas_call(
        _enc_proj_body,
        grid=(N // BN,),
        in_specs=[
            pl.BlockSpec((BN, H), lambda i: (i, 0)),
            wspec,
            pl.BlockSpec((1, H), lambda i: (0, 0)),
            wspec,
            wspec,
        ],
        out_specs=[pl.BlockSpec((BN, H), lambda i: (i, 0))] * 3,
        out_shape=[jax.ShapeDtypeStruct((N, H), f32)] * 3,
    )(x, W_enc, b_enc2, W1a, W1b)


def _message_body(g_ref, d_ref, ea_ref, W1cd_ref, b1_ref,
                  Wee_ref, bee_ref, W2_ref, b2_ref, m2_ref, cdn_ref):
    w1c = W1cd_ref[0:1, :]
    Wd = W1cd_ref[1:, :]
    Wfold = jnp.dot(Wee_ref[...], Wd, preferred_element_type=f32)
    b1f = b1_ref[...] + jnp.dot(bee_ref[...], Wd, preferred_element_type=f32)
    d = d_ref[...]
    dist = jnp.sqrt(jnp.sum(d * d, axis=1, keepdims=True))
    m1 = (g_ref[...] + dist * w1c
          + jnp.dot(ea_ref[...], Wfold, preferred_element_type=f32) + b1f)
    m = _silu(m1)
    m2_ref[...] = _silu(jnp.dot(m, W2_ref[...], preferred_element_type=f32)
                        + b2_ref[...])
    cdn_ref[...] = -d / (dist + 1e-8)


def _tc_message(g, d16, ea, W1cd, b1_2, We_enc, be_enc2, W2, b2_2):
    BE = 1000
    const = lambda shape: pl.BlockSpec(shape, lambda i: (0, 0))
    return pl.pallas_call(
        _message_body,
        grid=(E // BE,),
        in_specs=[
            pl.BlockSpec((BE, H), lambda i: (i, 0)),
            pl.BlockSpec((BE, PW), lambda i: (i, 0)),
            pl.BlockSpec((BE, DE), lambda i: (i, 0)),
            const((H + 1, H)),
            const((1, H)),
            const((DE, H)),
            const((1, H)),
            const((H, H)),
            const((1, H)),
        ],
        out_specs=[
            pl.BlockSpec((BE, H), lambda i: (i, 0)),
            pl.BlockSpec((BE, PW), lambda i: (i, 0)),
        ],
        out_shape=[
            jax.ShapeDtypeStruct((E, H), f32),
            jax.ShapeDtypeStruct((E, PW), f32),
        ],
    )(g, d16, ea, W1cd, b1_2, We_enc, be_enc2, W2, b2_2)


def _update_body(p0_ref, p1_ref, h_ref, nW1a_ref, nW1b_ref, nb1_ref,
                 nW2_ref, nb2_ref, cW1_ref, cb1_ref, cW2t_ref, cb2_ref,
                 W1a_ref, W1b_ref, hn_ref, hA_ref, hB_ref, cm_ref):
    agg = p0_ref[...] + p1_ref[...]
    u = _silu(jnp.dot(h_ref[...], nW1a_ref[...], preferred_element_type=f32)
              + jnp.dot(agg, nW1b_ref[...], preferred_element_type=f32)
              + nb1_ref[...])
    hn = jnp.dot(u, nW2_ref[...], preferred_element_type=f32) + nb2_ref[...]
    hn_ref[...] = hn
    hA_ref[...] = jnp.dot(hn, W1a_ref[...], preferred_element_type=f32)
    hB_ref[...] = jnp.dot(hn, W1b_ref[...], preferred_element_type=f32)
    t = _silu(jnp.dot(agg, cW1_ref[...], preferred_element_type=f32)
              + cb1_ref[...])
    cm = jnp.dot(t, cW2t_ref[...], preferred_element_type=f32) + cb2_ref[...]
    cm_ref[...] = cm[:, :PW]


def _update_last_body(p0_ref, p1_ref, h_ref, nW1a_ref, nW1b_ref, nb1_ref,
                      nW2_ref, nb2_ref, hn_ref):
    agg = p0_ref[...] + p1_ref[...]
    u = _silu(jnp.dot(h_ref[...], nW1a_ref[...], preferred_element_type=f32)
              + jnp.dot(agg, nW1b_ref[...], preferred_element_type=f32)
              + nb1_ref[...])
    hn_ref[...] = jnp.dot(u, nW2_ref[...], preferred_element_type=f32) + nb2_ref[...]


def _tc_update(parts, h, nW1a, nW1b, nb1_2, nW2, nb2_2,
               cW1, cb1_2, cW2t, cb2t, W1a, W1b):
    BN = 1000
    wspec = pl.BlockSpec((H, H), lambda i: (0, 0))
    bspec = pl.BlockSpec((1, H), lambda i: (0, 0))
    nspec = pl.BlockSpec((BN, H), lambda i: (i, 0))
    return pl.pallas_call(
        _update_body,
        grid=(N // BN,),
        in_specs=[
            pl.BlockSpec((BN, H), lambda i: (i, 0)),
            pl.BlockSpec((BN, H), lambda i: (i + N // BN, 0)),
            nspec, wspec, wspec, bspec, wspec, bspec,
            wspec, bspec, wspec, bspec, wspec, wspec,
        ],
        out_specs=[nspec, nspec, nspec, pl.BlockSpec((BN, PW), lambda i: (i, 0))],
        out_shape=[
            jax.ShapeDtypeStruct((N, H), f32),
            jax.ShapeDtypeStruct((N, H), f32),
            jax.ShapeDtypeStruct((N, H), f32),
            jax.ShapeDtypeStruct((N, PW), f32),
        ],
    )(parts, parts, h, nW1a, nW1b, nb1_2, nW2, nb2_2,
      cW1, cb1_2, cW2t, cb2t, W1a, W1b)


def _tc_update_last(parts, h, nW1a, nW1b, nb1_2, nW2, nb2_2):
    BN = 1000
    wspec = pl.BlockSpec((H, H), lambda i: (0, 0))
    bspec = pl.BlockSpec((1, H), lambda i: (0, 0))
    nspec = pl.BlockSpec((BN, H), lambda i: (i, 0))
    return pl.pallas_call(
        _update_last_body,
        grid=(N // BN,),
        in_specs=[
            pl.BlockSpec((BN, H), lambda i: (i, 0)),
            pl.BlockSpec((BN, H), lambda i: (i + N // BN, 0)),
            nspec, wspec, wspec, bspec, wspec, bspec,
        ],
        out_specs=nspec,
        out_shape=jax.ShapeDtypeStruct((N, H), f32),
    )(parts, parts, h, nW1a, nW1b, nb1_2, nW2, nb2_2)


def _readout_body(h_ref, W1_ref, b1_ref, w2_ref, b2_ref, out_ref):
    hg = jnp.mean(h_ref[...], axis=0, keepdims=True)
    hg8 = jnp.broadcast_to(hg, (8, H))
    r = jnp.maximum(
        jnp.dot(hg8, W1_ref[...], preferred_element_type=f32) + b1_ref[...], 0.0)
    o = jnp.sum(r * w2_ref[...], axis=1, keepdims=True) + b2_ref[...]
    out_ref[...] = o[0:1, :]


def _tc_readout(h, ro_W1, ro_b1_2, ro_w2r, ro_b2_2):
    return pl.pallas_call(
        _readout_body,
        out_shape=jax.ShapeDtypeStruct((1, 1), f32),
    )(h, ro_W1, ro_b1_2, ro_w2r, ro_b2_2)


def _stage_rows(s, src_hbm, dst_sh, buf):
    for k in range(RB // C):
        sl = pl.ds(s * RB + k * C, C)
        pltpu.sync_copy(src_hbm.at[sl], buf)
        pltpu.sync_copy(buf, dst_sh.at[sl])
    rem = RB - (RB // C) * C
    if rem:
        sl = pl.ds(s * RB + (RB // C) * C, rem)
        bsl = buf.at[pl.ds(0, rem)]
        pltpu.sync_copy(src_hbm.at[sl], bsl)
        pltpu.sync_copy(bsl, dst_sh.at[sl])

    @pl.when(s == NS - 1)
    def _():
        sl = pl.ds(RLS, RTAIL)
        bsl = buf.at[pl.ds(0, RTAIL)]
        pltpu.sync_copy(src_hbm.at[sl], bsl)
        pltpu.sync_copy(bsl, dst_sh.at[sl])


def _unstage_rows(s, src_sh, dst_hbm, buf, dst_off=0):
    for k in range(RB // C):
        sl = pl.ds(s * RB + k * C, C)
        pltpu.sync_copy(src_sh.at[sl], buf)
        pltpu.sync_copy(buf, dst_hbm.at[pl.ds(dst_off + s * RB + k * C, C)])
    rem = RB - (RB // C) * C
    if rem:
        sl = pl.ds(s * RB + (RB // C) * C, rem)
        bsl = buf.at[pl.ds(0, rem)]
        pltpu.sync_copy(src_sh.at[sl], bsl)
        pltpu.sync_copy(bsl, dst_hbm.at[pl.ds(dst_off + s * RB + (RB // C) * C, rem)])

    @pl.when(s == NS - 1)
    def _():
        bsl = buf.at[pl.ds(0, RTAIL)]
        pltpu.sync_copy(src_sh.at[pl.ds(RLS, RTAIL)], bsl)
        pltpu.sync_copy(bsl, dst_hbm.at[pl.ds(dst_off + RLS, RTAIL)])


def _sc_gpos_body(pos16, cdn, cmsg16, hA, hB, row, col,
                  g_out, d_out, pos_out,
                  idxr, idxc, idxs, bufA, bufB, bufPC, bufPR, bufc,
                  bufS, bufM, pos_sh, s_sh, sem):
    c = lax.axis_index("c")
    s = lax.axis_index("s")

    _stage_rows(s, pos16, pos_sh, bufPC)

    def zb(i, carry):
        bufc[i, :] = jnp.zeros((PW,), f32)
        return carry
    lax.fori_loop(0, CS, zb, 0)
    for k in range(RB // CS):
        pltpu.sync_copy(bufc, s_sh.at[pl.ds(s * RB + k * CS, CS)])
    remz = RB - (RB // CS) * CS
    pltpu.sync_copy(bufc.at[pl.ds(0, remz)],
                    s_sh.at[pl.ds(s * RB + (RB // CS) * CS, remz)])

    @pl.when(s == NS - 1)
    def _():
        pltpu.sync_copy(bufc.at[pl.ds(0, RTAIL)], s_sh.at[pl.ds(RLS, RTAIL)])

    plsc.subcore_barrier()

    def sbody(j, carry):
        base = s * EPT1 + j * CS
        pltpu.sync_copy(row.at[pl.ds(base, CS)], idxs)
        pltpu.sync_copy(cdn.at[pl.ds(base, CS)], bufc)
        pltpu.sync_copy(bufc, s_sh.at[idxs], add=True)
        return carry

    lax.fori_loop(0, EPT1 // CS, sbody, 0)
    plsc.subcore_barrier()

    def _apply(start, n):
        bsl = pl.ds(0, n)
        sl = pl.ds(start, n)
        pltpu.sync_copy(s_sh.at[sl], bufS.at[bsl])
        pltpu.sync_copy(cmsg16.at[sl], bufM.at[bsl])
        pltpu.sync_copy(pos_sh.at[sl], bufPC.at[bsl])

        def ab(i, carry):
            bufPC[i, :] = bufPC[i, :] + bufM[i, :] * bufS[i, :]
            return carry
        lax.fori_loop(0, n, ab, 0)
        pltpu.sync_copy(bufPC.at[bsl], pos_sh.at[sl])

        @pl.when(c == 0)
        def _():
            pltpu.sync_copy(bufPC.at[bsl], pos_out.at[sl])

    for k in range(RB // C):
        _apply(s * RB + k * C, C)
    _apply(s * RB + (RB // C) * C, RB - (RB // C) * C)

    @pl.when(s == NS - 1)
    def _():
        _apply(RLS, RTAIL)

    plsc.subcore_barrier()

    base_e = (c * NS + s) * EPT

    def gbody(j, carry):
        base = base_e + j * C
        pltpu.sync_copy(row.at[pl.ds(base, C)], idxr)
        pltpu.sync_copy(col.at[pl.ds(base, C)], idxc)
        cpA = pltpu.async_copy(hA.at[idxc], bufA, sem)
        cpB = pltpu.async_copy(hB.at[idxr], bufB, sem)
        pltpu.sync_copy(pos_sh.at[idxc], bufPC)
        pltpu.sync_copy(pos_sh.at[idxr], bufPR)
        cpA.wait()
        cpB.wait()

        def rbody(i, carry2):
            bufPC[i, :] = bufPC[i, :] - bufPR[i, :]
            for k in range(H // 16):
                sl = pl.ds(k * 16, 16)
                bufA[i, sl] = bufA[i, sl] + bufB[i, sl]
            return carry2
        lax.fori_loop(0, C, rbody, 0)

        pltpu.sync_copy(bufA, g_out.at[pl.ds(base, C)])
        pltpu.sync_copy(bufPC, d_out.at[pl.ds(base, C)])
        return carry

    lax.fori_loop(0, EPT // C, gbody, 0)


_sc_gpos = functools.partial(
    pl.kernel,
    out_type=[
        jax.ShapeDtypeStruct((E, H), f32),
        jax.ShapeDtypeStruct((E, PW), f32),
        jax.ShapeDtypeStruct((N, PW), f32),
    ],
    mesh=_MESH,
    scratch_types=[
        pltpu.VMEM((C,), jnp.int32),
        pltpu.VMEM((C,), jnp.int32),
        pltpu.VMEM((CS,), jnp.int32),
        pltpu.VMEM((C, H), f32),
        pltpu.VMEM((C, H), f32),
        pltpu.VMEM((C, PW), f32),
        pltpu.VMEM((C, PW), f32),
        pltpu.VMEM((CS, PW), f32),
        pltpu.VMEM((C, PW), f32),
        pltpu.VMEM((C, PW), f32),
        pltpu.VMEM_SHARED((N, PW), f32),
        pltpu.VMEM_SHARED((N, PW), f32),
        pltpu.SemaphoreType.DMA,
    ],
)(_sc_gpos_body)


def _sc_segsum_body(m2, col, parts_out, idxb, buf, zbuf, agg_sh, sem):
    c = lax.axis_index("c")
    s = lax.axis_index("s")

    def zbody(i, carry):
        def zcol(k, carry2):
            zbuf[i, pl.ds(k * 16, 16)] = jnp.zeros((16,), f32)
            return carry2
        return lax.fori_loop(0, H // 16, zcol, carry)
    lax.fori_loop(0, RTAIL, zbody, 0)
    for k in range(RB // RTAIL):
        pltpu.sync_copy(zbuf, agg_sh.at[pl.ds(s * RB + k * RTAIL, RTAIL)])

    @pl.when(s == NS - 1)
    def _():
        pltpu.sync_copy(zbuf, agg_sh.at[pl.ds(RLS, RTAIL)])

    plsc.subcore_barrier()
    base_e = (c * NS + s) * EPT

    def body(j, carry):
        base = base_e + j * C
        pltpu.sync_copy(col.at[pl.ds(base, C)], idxb)
        pltpu.sync_copy(m2.at[pl.ds(base, C)], buf)
        pltpu.sync_copy(buf, agg_sh.at[idxb], add=True)
        return carry

    lax.fori_loop(0, EPT // C, body, 0)
    plsc.subcore_barrier()
    _unstage_rows(s, agg_sh, parts_out, buf, dst_off=c * N)


_sc_segsum = functools.partial(
    pl.kernel,
    out_type=jax.ShapeDtypeStruct((2 * N, H), f32),
    mesh=_MESH,
    scratch_types=[
        pltpu.VMEM((C,), jnp.int32),
        pltpu.VMEM((C, H), f32),
        pltpu.VMEM((RTAIL, H), f32),
        pltpu.VMEM_SHARED((N, H), f32),
        pltpu.SemaphoreType.DMA,
    ],
)(_sc_segsum_body)


def kernel(x, pos, edge_index, edge_attr, W_enc, b_enc, We_enc, be_enc,
           msg_W1, msg_b1, msg_W2, msg_b2, coord_W1, coord_b1, coord_W2,
           coord_b2, node_W1, node_b1, node_W2, node_b2,
           ro_W1, ro_b1, ro_W2, ro_b2):
    row = edge_index[0]
    col = edge_index[1]
    pos16 = jnp.pad(pos, ((0, 0), (0, PW - 3)))
    be_enc2 = be_enc.reshape(1, H)

    h, hA, hB = _tc_encode_proj(x, W_enc, b_enc.reshape(1, H),
                                msg_W1[0, :H], msg_W1[0, H:2 * H])

    cdn_prev = jnp.zeros((E, PW), f32)
    cmsg_prev = jnp.zeros((N, PW), f32)

    for l in range(L):
        g, d16, pos16 = _sc_gpos(pos16, cdn_prev, cmsg_prev, hA, hB, row, col)
        m2, cdn = _tc_message(g, d16, edge_attr,
                              msg_W1[l, 2 * H:], msg_b1[l].reshape(1, H),
                              We_enc, be_enc2,
                              msg_W2[l], msg_b2[l].reshape(1, H))
        parts = _sc_segsum(m2, col)
        if l < L - 1:
            h, hA, hB, cm16 = _tc_update(
                parts, h,
                node_W1[l, :H], node_W1[l, H:], node_b1[l].reshape(1, H),
                node_W2[l], node_b2[l].reshape(1, H),
                coord_W1[l], coord_b1[l].reshape(1, H),
                jnp.tile(coord_W2[l], (1, H)),
                jnp.tile(coord_b2[l].reshape(1, 1), (1, H)),
                msg_W1[l + 1, :H], msg_W1[l + 1, H:2 * H])
            cdn_prev, cmsg_prev = cdn, cm16
        else:
            h = _tc_update_last(
                parts, h,
                node_W1[l, :H], node_W1[l, H:], node_b1[l].reshape(1, H),
                node_W2[l], node_b2[l].reshape(1, H))

    return _tc_readout(h, ro_W1, ro_b1.reshape(1, H),
                       ro_W2.reshape(1, H), ro_b2.reshape(1, 1))
